# Initial kernel scaffold; baseline (speedup 1.0000x reference)
import jax
import jax.numpy as jnp
from jax.experimental import pallas as pl


def kernel(x, Wq, bq, Wk, bk, Wv, bv, Wo, bo, gW, gb, eW1, eb1, eW2, eb2, g1, b1n, g2, b2n):
    raise NotImplementedError("write your pallas kernel here")


# speedup vs baseline: 1.4744x; 1.4744x over previous
import functools
from math import sqrt

import jax
import jax.numpy as jnp
from jax.experimental import pallas as pl
from jax.experimental.pallas import tpu as pltpu

B, S, D = 1, 2048, 1024
H = 16
HD = D // H
E = 8
TOPK = 2
DFF = 2048


def _qkv_kernel(x_ref, w_ref, b_ref, out_ref):
    acc = jnp.dot(x_ref[...], w_ref[...], preferred_element_type=jnp.float32)
    out_ref[...] = acc + b_ref[...]


def _qkv_proj(x2, wqkv, bqkv):
    BM, BN = 512, 512
    return pl.pallas_call(
        _qkv_kernel,
        grid=(S // BM, 3 * D // BN),
        in_specs=[
            pl.BlockSpec((BM, D), lambda i, j: (i, 0)),
            pl.BlockSpec((D, BN), lambda i, j: (0, j)),
            pl.BlockSpec((1, BN), lambda i, j: (0, j)),
        ],
        out_specs=pl.BlockSpec((BM, BN), lambda i, j: (i, j)),
        out_shape=jax.ShapeDtypeStruct((S, 3 * D), jnp.float32),
    )(x2, wqkv, bqkv)


def _rope(u, cos, sin):
    u1 = u[:, :HD // 2]
    u2 = u[:, HD // 2:]
    rot = jnp.concatenate([-u2, u1], axis=1)
    return u * cos + rot * sin


def _attn_kernel_v2(q_ref, k_ref, v_ref, cos_ref, sin_ref, out_ref, *, bq):
    i = pl.program_id(1)
    q = q_ref[0]
    k = k_ref[0]
    v = v_ref[0]
    cos_q = cos_ref[pl.ds(i * bq, bq), :]
    sin_q = sin_ref[pl.ds(i * bq, bq), :]
    q = _rope(q, cos_q, sin_q)
    k = _rope(k, cos_ref[...], sin_ref[...])
    scores = jax.lax.dot_general(
        q, k, (((1,), (1,)), ((), ())),
        preferred_element_type=jnp.float32) * (1.0 / sqrt(HD))
    m = jnp.max(scores, axis=-1, keepdims=True)
    p = jnp.exp(scores - m)
    l = jnp.sum(p, axis=-1, keepdims=True)
    o = jnp.dot(p, v, preferred_element_type=jnp.float32)
    out_ref[0] = o / l


def _attention_v2(qkv_h, cos, sin):
    BQ = 512
    kern = functools.partial(_attn_kernel_v2, bq=BQ)
    return pl.pallas_call(
        kern,
        grid=(H, S // BQ),
        in_specs=[
            pl.BlockSpec((1, BQ, HD), lambda h, i: (h, i, 0)),
            pl.BlockSpec((1, S, HD), lambda h, i: (H + h, 0, 0)),
            pl.BlockSpec((1, S, HD), lambda h, i: (2 * H + h, 0, 0)),
            pl.BlockSpec((S, HD), lambda h, i: (0, 0)),
            pl.BlockSpec((S, HD), lambda h, i: (0, 0)),
        ],
        out_specs=pl.BlockSpec((1, BQ, HD), lambda h, i: (h, i, 0)),
        out_shape=jax.ShapeDtypeStruct((H, S, HD), jnp.float32),
    )(qkv_h, qkv_h, qkv_h, cos, sin)


def _proj_ln_gate_kernel(ao_ref, wo_ref, bo_ref, x_ref, g1_ref, b1_ref,
                         gw_ref, gb_ref, x1_ref, w_ref):
    t = jnp.dot(ao_ref[...], wo_ref[...], preferred_element_type=jnp.float32)
    t = t + bo_ref[...] + x_ref[...]
    m = jnp.mean(t, axis=-1, keepdims=True)
    c = t - m
    v = jnp.mean(c * c, axis=-1, keepdims=True)
    x1 = c * jax.lax.rsqrt(v + 1e-5) * g1_ref[...] + b1_ref[...]
    x1_ref[...] = x1
    logits = jnp.dot(x1, gw_ref[...], preferred_element_type=jnp.float32) + gb_ref[...]
    lm = jnp.max(logits, axis=-1, keepdims=True)
    pe = jnp.exp(logits - lm)
    probs = pe / jnp.sum(pe, axis=-1, keepdims=True)
    lane = jax.lax.broadcasted_iota(jnp.int32, probs.shape, 1)
    m1 = jnp.max(probs, axis=-1, keepdims=True)
    i1 = jnp.min(jnp.where(probs == m1, lane, E), axis=-1, keepdims=True)
    first1 = lane == i1
    p2 = jnp.where(first1, -jnp.inf, probs)
    m2 = jnp.max(p2, axis=-1, keepdims=True)
    i2 = jnp.min(jnp.where(p2 == m2, lane, E), axis=-1, keepdims=True)
    first2 = lane == i2
    denom = m1 + m2
    w = (first1 * m1 + first2 * m2) / denom
    w_ref[...] = w.astype(jnp.float32)


def _proj_ln_gate(attn_o, wo, bo, x2, g1, b1n, gw, gb):
    BM = 512
    return pl.pallas_call(
        _proj_ln_gate_kernel,
        grid=(S // BM,),
        in_specs=[
            pl.BlockSpec((BM, D), lambda i: (i, 0)),
            pl.BlockSpec((D, D), lambda i: (0, 0)),
            pl.BlockSpec((1, D), lambda i: (0, 0)),
            pl.BlockSpec((BM, D), lambda i: (i, 0)),
            pl.BlockSpec((1, D), lambda i: (0, 0)),
            pl.BlockSpec((1, D), lambda i: (0, 0)),
            pl.BlockSpec((D, E), lambda i: (0, 0)),
            pl.BlockSpec((1, E), lambda i: (0, 0)),
        ],
        out_specs=[
            pl.BlockSpec((BM, D), lambda i: (i, 0)),
            pl.BlockSpec((BM, E), lambda i: (i, 0)),
        ],
        out_shape=[
            jax.ShapeDtypeStruct((S, D), jnp.float32),
            jax.ShapeDtypeStruct((S, E), jnp.float32),
        ],
    )(attn_o, wo, bo, x2, g1, b1n, gw, gb)


def _moe_kernel(x1_ref, w_ref, ew1_ref, eb1_ref, ew2_ref, eb2_ref, out_ref):
    e = pl.program_id(1)

    @pl.when(e == 0)
    def _():
        out_ref[...] = jnp.zeros_like(out_ref)

    h = jnp.dot(x1_ref[...], ew1_ref[0], preferred_element_type=jnp.float32)
    h = h + eb1_ref[0]
    h = 0.5 * h * (1.0 + jax.lax.erf(h * 0.7071067811865476))
    oe = jnp.dot(h, ew2_ref[0], preferred_element_type=jnp.float32) + eb2_ref[0]
    lane = jax.lax.broadcasted_iota(jnp.int32, (1, E), 1)
    sel = jnp.sum(jnp.where(lane == e, w_ref[...], 0.0), axis=-1, keepdims=True)
    out_ref[...] += sel * oe


def _moe_dense(x1, w, ew1, eb1, ew2, eb2):
    BM = 512
    return pl.pallas_call(
        _moe_kernel,
        grid=(S // BM, E),
        in_specs=[
            pl.BlockSpec((BM, D), lambda i, e: (i, 0)),
            pl.BlockSpec((BM, E), lambda i, e: (i, 0)),
            pl.BlockSpec((1, D, DFF), lambda i, e: (e, 0, 0)),
            pl.BlockSpec((1, 1, DFF), lambda i, e: (e, 0, 0)),
            pl.BlockSpec((1, DFF, D), lambda i, e: (e, 0, 0)),
            pl.BlockSpec((1, 1, D), lambda i, e: (e, 0, 0)),
        ],
        out_specs=pl.BlockSpec((BM, D), lambda i, e: (i, 0)),
        out_shape=jax.ShapeDtypeStruct((S, D), jnp.float32),
        compiler_params=pltpu.CompilerParams(
            dimension_semantics=("parallel", "arbitrary")),
    )(x1, w, ew1, eb1.reshape(E, 1, DFF), ew2, eb2.reshape(E, 1, D))


def _final_ln_kernel(x1_ref, moe_ref, g_ref, b_ref, out_ref):
    t = x1_ref[...] + moe_ref[...]
    m = jnp.mean(t, axis=-1, keepdims=True)
    c = t - m
    v = jnp.mean(c * c, axis=-1, keepdims=True)
    out_ref[...] = c * jax.lax.rsqrt(v + 1e-5) * g_ref[...] + b_ref[...]


def _final_ln(x1, moe, g2, b2n):
    BM = 512
    return pl.pallas_call(
        _final_ln_kernel,
        grid=(S // BM,),
        in_specs=[
            pl.BlockSpec((BM, D), lambda i: (i, 0)),
            pl.BlockSpec((BM, D), lambda i: (i, 0)),
            pl.BlockSpec((1, D), lambda i: (0, 0)),
            pl.BlockSpec((1, D), lambda i: (0, 0)),
        ],
        out_specs=pl.BlockSpec((BM, D), lambda i: (i, 0)),
        out_shape=jax.ShapeDtypeStruct((S, D), jnp.float32),
    )(x1, moe, g2, b2n)


def kernel(x, Wq, bq, Wk, bk, Wv, bv, Wo, bo, gW, gb, eW1, eb1, eW2, eb2,
           g1, b1n, g2, b2n):
    x2 = x.reshape(S, D)
    wqkv = jnp.concatenate([Wq, Wk, Wv], axis=1)
    bqkv = jnp.concatenate([bq, bk, bv]).reshape(1, 3 * D)

    qkv = _qkv_proj(x2, wqkv, bqkv)
    qkv_h = qkv.reshape(S, 3 * H, HD).transpose(1, 0, 2)

    inv_freq = 1.0 / (10000.0 ** (jnp.arange(0, HD, 2, dtype=jnp.float32) / HD))
    t = jnp.arange(S, dtype=jnp.float32)
    freqs = t[:, None] * inv_freq[None, :]
    emb = jnp.concatenate((freqs, freqs), axis=-1)
    cos = jnp.cos(emb)
    sin = jnp.sin(emb)

    attn = _attention_v2(qkv_h, cos, sin)
    attn_o = attn.transpose(1, 0, 2).reshape(S, D)

    x1, w = _proj_ln_gate(attn_o, Wo, bo.reshape(1, D), x2,
                          g1.reshape(1, D), b1n.reshape(1, D),
                          gW, gb.reshape(1, E))

    moe = _moe_dense(x1, w, eW1, eb1, eW2, eb2)

    out = _final_ln(x1, moe, g2.reshape(1, D), b2n.reshape(1, D))
    return out.reshape(B, S, D)
